# baseline (speedup 1.0000x reference)
import jax
import jax.numpy as jnp
from jax import lax
from jax.experimental import pallas as pl
from jax.experimental.pallas import tpu as pltpu
from jax.experimental.pallas import tpu_sc as plsc

EMB_DIM = 64
BATCH = 16384

_info = plsc.get_sparse_core_info()
_NC, _NS, _L = _info.num_cores, _info.num_subcores, _info.num_lanes
_NW = _NC * _NS
_BPW = BATCH // _NW
_SPR = EMB_DIM // _L
_NSL = _BPW * _SPR


def _sc_body(sbj_hbm, idx_hbm, wr_hbm, out_hbm, idx_v, idx2_v, rows_v, sbj_v,
             out_v, pscr_v, sem_g, sem_s):
    wid = lax.axis_index("s") * _NC + lax.axis_index("c")
    base = wid * _BPW
    pltpu.sync_copy(idx_hbm.at[pl.ds(base, _BPW)], idx_v)
    cps = pltpu.async_copy(sbj_hbm.at[pl.ds(base * EMB_DIM, _BPW * EMB_DIM)],
                           sbj_v, sem_s)

    lane = lax.iota(jnp.int32, _L)

    def expand(m, carry):
        pos = m * _L + lane
        rows = plsc.load_gather(idx_v, [pos >> 2])
        idx2_v[pl.ds(m * _L, _L)] = rows * _SPR + (pos & (_SPR - 1))
        return carry

    lax.fori_loop(0, _NSL // _L, expand, 0)

    gat = pltpu.async_copy(wr_hbm.at[idx2_v], rows_v, sem_g)
    gat.wait()
    cps.wait()

    def group(g, carry):
        for jj in range(_L):
            j = g * _L + jj
            acc = jnp.zeros((_L,), jnp.float32)
            for c in range(_SPR):
                s = sbj_v[pl.ds(j * EMB_DIM + c * _L, _L)]
                r = rows_v[j * _SPR + c, :]
                acc = acc + s * (r * r)
            pscr_v[pl.ds(jj * _L, _L)] = acc
        tot = jnp.zeros((_L,), jnp.float32)
        for d in range(_L):
            col = plsc.load_gather(pscr_v, [lane * _L + d])
            tot = tot + col
        out_v[pl.ds(g * _L, _L)] = tot
        return carry

    lax.fori_loop(0, _BPW // _L, group, 0)
    pltpu.sync_copy(out_v, out_hbm.at[pl.ds(base, _BPW)])


def kernel(sbj_embs, obj_embs, rel_ids, W_r):
    mesh = plsc.VectorSubcoreMesh(core_axis_name="c", subcore_axis_name="s")
    k = pl.kernel(
        _sc_body,
        mesh=mesh,
        compiler_params=pltpu.CompilerParams(
            needs_layout_passes=False, use_tc_tiling_on_sc=False),
        out_type=jax.ShapeDtypeStruct((BATCH,), jnp.float32),
        scratch_types=[
            pltpu.VMEM((_BPW,), jnp.int32),
            pltpu.VMEM((_NSL,), jnp.int32),
            pltpu.VMEM((_NSL, _L), jnp.float32),
            pltpu.VMEM((_BPW * EMB_DIM,), jnp.float32),
            pltpu.VMEM((_BPW,), jnp.float32),
            pltpu.VMEM((_L * _L,), jnp.float32),
            pltpu.SemaphoreType.DMA,
            pltpu.SemaphoreType.DMA,
        ],
    )
    return k(sbj_embs.reshape(-1), rel_ids.astype(jnp.int32),
             W_r.reshape(-1, _L))


# speedup vs baseline: 1.5291x; 1.5291x over previous
import jax
import jax.numpy as jnp
from jax import lax
from jax.experimental import pallas as pl
from jax.experimental.pallas import tpu as pltpu
from jax.experimental.pallas import tpu_sc as plsc

EMB_DIM = 64
BATCH = 16384
VOCAB = 100000

_info = plsc.get_sparse_core_info()
_NC, _NS, _L = _info.num_cores, _info.num_subcores, _info.num_lanes
_HB = BATCH // _NC
_TS = _HB // _NS
_CHUNKS = _HB // _L
_WAVES = EMB_DIM // _NS


def _sc_body(sbjT_hbm, idx_hbm, wrT_hbm, out_hbm, part_hbm,
             col_v, idx_v, sbj_v, acc_v, rbuf_v, racc_v, sem):
    s = lax.axis_index("c")
    t = lax.axis_index("s")
    base = s * _HB
    pltpu.sync_copy(idx_hbm.at[pl.ds(base, _HB)], idx_v)

    for wave in range(_WAVES):
        d = wave * _NS + t
        pltpu.sync_copy(wrT_hbm.at[d], col_v)
        pltpu.sync_copy(sbjT_hbm.at[d, pl.ds(base, _HB)], sbj_v)

        def chunk(m, carry):
            sl = pl.ds(m * _L, _L)
            i16 = idx_v[sl]
            w16 = plsc.load_gather(col_v, [i16])
            c16 = sbj_v[sl] * (w16 * w16)
            if wave == 0:
                acc_v[sl] = c16
            else:
                acc_v[sl] = acc_v[sl] + c16
            return carry

        lax.fori_loop(0, _CHUNKS, chunk, 0)

    pltpu.sync_copy(acc_v, part_hbm.at[s, t])
    plsc.subcore_barrier()

    seg = pl.ds(t * _TS, _TS)
    for half in range(2):
        cps = [
            pltpu.async_copy(part_hbm.at[s, half * 8 + p, seg],
                             rbuf_v.at[p], sem)
            for p in range(8)
        ]
        for cp in cps:
            cp.wait()

        def red(m, carry):
            sl = pl.ds(m * _L, _L)
            v = rbuf_v[0, sl]
            for p in range(1, 8):
                v = v + rbuf_v[p, sl]
            if half == 0:
                racc_v[sl] = v
            else:
                racc_v[sl] = racc_v[sl] + v
            return carry

        lax.fori_loop(0, _TS // _L, red, 0)

    pltpu.sync_copy(racc_v, out_hbm.at[pl.ds(base + t * _TS, _TS)])


def kernel(sbj_embs, obj_embs, rel_ids, W_r):
    mesh = plsc.VectorSubcoreMesh(core_axis_name="c", subcore_axis_name="s")
    k = pl.kernel(
        _sc_body,
        mesh=mesh,
        compiler_params=pltpu.CompilerParams(
            needs_layout_passes=False, use_tc_tiling_on_sc=True),
        out_type=(
            jax.ShapeDtypeStruct((BATCH,), jnp.float32),
            jax.ShapeDtypeStruct((_NC, _NS, _HB), jnp.float32),
        ),
        scratch_types=[
            pltpu.VMEM((VOCAB,), jnp.float32),
            pltpu.VMEM((_HB,), jnp.int32),
            pltpu.VMEM((_HB,), jnp.float32),
            pltpu.VMEM((_HB,), jnp.float32),
            pltpu.VMEM((8, _TS), jnp.float32),
            pltpu.VMEM((_TS,), jnp.float32),
            pltpu.SemaphoreType.DMA,
        ],
    )
    out, _ = k(sbj_embs.T, rel_ids.astype(jnp.int32), W_r.T)
    return out
